# Initial kernel scaffold; baseline (speedup 1.0000x reference)
import jax
import jax.numpy as jnp
from jax.experimental import pallas as pl


def kernel(x, edge_index, W1, b1, W2, b2):
    raise NotImplementedError("write your pallas kernel here")


# speedup vs baseline: 11.9168x; 11.9168x over previous
import functools

import jax
import jax.numpy as jnp
from jax import lax
from jax.experimental import pallas as pl
from jax.experimental.pallas import tpu as pltpu
from jax.experimental.pallas import tpu_sc as plsc

N_NODES = 10000
N_EDGES = 320000
D = 128

NC = 2
NS = 16
NW = NC * NS
NP = 10240
RPT = NP // NS
CH = 80
EPW = N_EDGES // NW
EPT = N_EDGES // NS

_mesh = plsc.VectorSubcoreMesh(core_axis_name="c", subcore_axis_name="s")


@functools.partial(
    pl.kernel,
    out_type=jax.ShapeDtypeStruct((NP,), jnp.float32),
    mesh=_mesh,
    scratch_types=[
        pltpu.VMEM((CH,), jnp.int32),
        pltpu.VMEM((CH,), jnp.float32),
        pltpu.VMEM((RPT,), jnp.float32),
        pltpu.VMEM_SHARED((NP,), jnp.float32),
    ],
)
def _deg(dst_hbm, zeros1_hbm, ones_hbm, deg_hbm, dstv, onesv, degv, deg_sh):
    c = lax.axis_index("c")
    s = lax.axis_index("s")
    base_r = s * RPT
    pltpu.sync_copy(zeros1_hbm, degv)
    pltpu.sync_copy(degv, deg_sh.at[pl.ds(base_r, RPT)])
    pltpu.sync_copy(ones_hbm, onesv)
    plsc.subcore_barrier()

    base_e = s * EPT

    def chunk(i, carry):
        off = base_e + i * CH
        pltpu.sync_copy(dst_hbm.at[pl.ds(off, CH)], dstv)
        pltpu.sync_copy(onesv, deg_sh.at[dstv], add=True)
        return carry

    lax.fori_loop(0, EPT // CH, chunk, 0)
    plsc.subcore_barrier()

    pltpu.sync_copy(deg_sh.at[pl.ds(base_r, RPT)], degv)

    @pl.when(c == 0)
    def _():
        pltpu.sync_copy(degv, deg_hbm.at[pl.ds(base_r, RPT)])


@functools.partial(
    pl.kernel,
    out_type=jax.ShapeDtypeStruct((NC, NP, D), jnp.float32),
    mesh=_mesh,
    scratch_types=[
        pltpu.VMEM((CH,), jnp.int32),
        pltpu.VMEM((CH,), jnp.int32),
        pltpu.VMEM((CH, D), jnp.float32),
        pltpu.VMEM_SHARED((NP, D), jnp.float32),
        pltpu.SemaphoreType.DMA,
    ],
)
def _gather_scatter(g_hbm, src_hbm, dst_hbm, zeros2_hbm, out_hbm,
                    srcv, dstv, rows, acc_sh, sem):
    c = lax.axis_index("c")
    s = lax.axis_index("s")
    base_r = s * RPT
    pltpu.sync_copy(zeros2_hbm, acc_sh.at[pl.ds(base_r, RPT)])
    plsc.subcore_barrier()

    base_e = (s * NC + c) * EPW

    def chunk(i, carry):
        off = base_e + i * CH
        pltpu.sync_copy(src_hbm.at[pl.ds(off, CH)], srcv)
        pltpu.sync_copy(dst_hbm.at[pl.ds(off, CH)], dstv)
        pltpu.async_copy(g_hbm.at[srcv], rows, sem).wait()
        pltpu.sync_copy(rows, acc_sh.at[dstv], add=True)
        return carry

    lax.fori_loop(0, EPW // CH, chunk, 0)
    plsc.subcore_barrier()

    pltpu.sync_copy(acc_sh.at[pl.ds(base_r, RPT)], out_hbm.at[c, pl.ds(base_r, RPT)])


_RB = 2000
_NB = N_NODES // _RB

_row_spec = pl.BlockSpec((_RB, D), lambda i: (i, 0))
_col_spec = pl.BlockSpec((_RB, 1), lambda i: (i, 0))
_w_spec = pl.BlockSpec((D, D), lambda i: (0, 0))
_b_spec = pl.BlockSpec((1, D), lambda i: (0, 0))


def _mm_scale_body(x_ref, w_ref, deg_ref, g_ref, dinv_ref):
    dinv = lax.rsqrt(deg_ref[...] + 1.0)
    dinv_ref[...] = dinv
    h = jnp.dot(x_ref[...], w_ref[...], preferred_element_type=jnp.float32)
    g_ref[...] = h * dinv


_mm_scale = pl.pallas_call(
    _mm_scale_body,
    grid=(_NB,),
    in_specs=[_row_spec, _w_spec, _col_spec],
    out_specs=(_row_spec, _col_spec),
    out_shape=(
        jax.ShapeDtypeStruct((N_NODES, D), jnp.float32),
        jax.ShapeDtypeStruct((N_NODES, 1), jnp.float32),
    ),
)


def _layer2_body(p0_ref, p1_ref, g1_ref, dinv_ref, w_ref, b_ref, g2_ref):
    pre = dinv_ref[...] * (p0_ref[...] + p1_ref[...] + g1_ref[...]) + b_ref[...]
    x1 = pre * 0.5 * (1.0 + lax.erf(pre * 0.7071067811865476))
    h2 = jnp.dot(x1, w_ref[...], preferred_element_type=jnp.float32)
    g2_ref[...] = h2 * dinv_ref[...]


_layer2 = pl.pallas_call(
    _layer2_body,
    grid=(_NB,),
    in_specs=[_row_spec, _row_spec, _row_spec, _col_spec, _w_spec, _b_spec],
    out_specs=_row_spec,
    out_shape=jax.ShapeDtypeStruct((N_NODES, D), jnp.float32),
)


def _final_body(q0_ref, q1_ref, g2_ref, dinv_ref, b_ref, out_ref):
    out_ref[...] = (
        dinv_ref[...] * (q0_ref[...] + q1_ref[...] + g2_ref[...]) + b_ref[...]
    )


_final = pl.pallas_call(
    _final_body,
    grid=(_NB,),
    in_specs=[_row_spec, _row_spec, _row_spec, _col_spec, _b_spec],
    out_specs=_row_spec,
    out_shape=jax.ShapeDtypeStruct((N_NODES, D), jnp.float32),
)


def kernel(x, edge_index, W1, b1, W2, b2):
    ei = edge_index.astype(jnp.int32)
    src = ei[0]
    dst = ei[1]
    zeros1 = jnp.zeros((RPT,), jnp.float32)
    ones = jnp.ones((CH,), jnp.float32)
    zeros2 = jnp.zeros((RPT, D), jnp.float32)

    deg_p = _deg(dst, zeros1, ones)
    deg = deg_p[:N_NODES].reshape(N_NODES, 1)

    g1, dinv = _mm_scale(x, W1, deg)
    acc1 = _gather_scatter(g1, src, dst, zeros2)
    g2 = _layer2(acc1[0, :N_NODES], acc1[1, :N_NODES], g1, dinv,
                 W2, b1.reshape(1, D))
    acc2 = _gather_scatter(g2, src, dst, zeros2)
    out = _final(acc2[0, :N_NODES], acc2[1, :N_NODES], g2, dinv,
                 b2.reshape(1, D))
    return out
